# Initial kernel scaffold; baseline (speedup 1.0000x reference)
import jax
import jax.numpy as jnp
from jax.experimental import pallas as pl


def kernel(x, edge_index, W1, b1, W2, b2, W3, b3):
    raise NotImplementedError("write your pallas kernel here")


# speedup vs baseline: 30.4223x; 30.4223x over previous
import functools

import jax
import jax.numpy as jnp
from jax import lax
from jax.experimental import pallas as pl
from jax.experimental.pallas import tpu as pltpu
from jax.experimental.pallas import tpu_sc as plsc

F32 = jnp.float32

N = 10000
E = 320000
D = 128

NC = 2
NS = 16
NW = NC * NS
K = 128
NPAD = 10240
TPW = NPAD // NS
CW = -(-E // (NW * K))
EPW = CW * K

_mesh = plsc.VectorSubcoreMesh(core_axis_name="c", subcore_axis_name="s")


def _zero_rows(buf):
    z = jnp.zeros((16,), F32)

    def body(i, _):
        for k in range(buf.shape[1] // 16):
            buf[i, pl.ds(k * 16, 16)] = z
        return 0

    lax.fori_loop(0, buf.shape[0], body, 0)


def _zero_1d(buf):
    z = jnp.zeros((16,), F32)

    def body(i, _):
        buf[pl.ds(i * 16, 16)] = z
        return 0

    lax.fori_loop(0, buf.shape[0] // 16, body, 0)


@functools.partial(
    pl.kernel,
    out_type=jax.ShapeDtypeStruct((NC, NPAD), F32),
    mesh=_mesh,
    scratch_types=[
        pltpu.VMEM((CW, K), jnp.int32),
        pltpu.VMEM((K,), F32),
        pltpu.VMEM((TPW,), F32),
        pltpu.VMEM_SHARED((NPAD,), F32),
    ],
)
def _deg(colw_hbm, out_hbm, colv, ones, zbuf, dacc):
    c = lax.axis_index("c")
    s = lax.axis_index("s")
    wid = s * NC + c
    pltpu.sync_copy(colw_hbm.at[wid], colv)
    for k in range(K // 16):
        ones[pl.ds(k * 16, 16)] = jnp.ones((16,), F32)
    _zero_1d(zbuf)
    pltpu.sync_copy(zbuf, dacc.at[pl.ds(s * TPW, TPW)])
    plsc.subcore_barrier()

    def body(j, _):
        pltpu.sync_copy(ones, dacc.at[colv.at[j]], add=True)
        return 0

    lax.fori_loop(0, CW, body, 0)
    plsc.subcore_barrier()
    pltpu.sync_copy(dacc.at[pl.ds(s * TPW, TPW)], out_hbm.at[c, pl.ds(s * TPW, TPW)])


@functools.partial(
    pl.kernel,
    out_type=jax.ShapeDtypeStruct((NC, NPAD, D), F32),
    mesh=_mesh,
    scratch_types=[
        pltpu.VMEM((2, 2, K), jnp.int32),
        pltpu.VMEM((K, D), F32),
        pltpu.VMEM((K, D), F32),
        pltpu.VMEM_SHARED((NPAD, D), F32),
        pltpu.SemaphoreType.DMA,
        pltpu.SemaphoreType.DMA,
        pltpu.SemaphoreType.DMA,
        pltpu.SemaphoreType.DMA,
    ],
)
def _agg(g_hbm, idx_hbm, out_hbm, ibuf, buf0, buf1, acc, sem0, sem1, is0, is1):
    c = lax.axis_index("c")
    s = lax.axis_index("s")
    wid = s * NC + c
    _zero_rows(buf0)
    for b in range(TPW // K):
        pltpu.sync_copy(buf0, acc.at[pl.ds(s * TPW + b * K, K)])
    plsc.subcore_barrier()

    pltpu.async_copy(idx_hbm.at[wid, 0], ibuf.at[0], is0)
    pltpu.async_copy(idx_hbm.at[wid, 1], ibuf.at[1], is1)
    pltpu.make_async_copy(idx_hbm.at[wid, 0], ibuf.at[0], is0).wait()
    pltpu.async_copy(g_hbm.at[ibuf.at[0, 0]], buf0, sem0)

    def body(k, _):
        j0 = 2 * k
        pltpu.make_async_copy(idx_hbm.at[wid, j0 + 1], ibuf.at[1], is1).wait()
        pltpu.async_copy(g_hbm.at[ibuf.at[1, 0]], buf1, sem1)
        pltpu.make_async_copy(g_hbm.at[ibuf.at[0, 0]], buf0, sem0).wait()
        pltpu.sync_copy(buf0, acc.at[ibuf.at[0, 1]], add=True)

        @pl.when(j0 + 2 < CW)
        def _():
            pltpu.async_copy(idx_hbm.at[wid, j0 + 2], ibuf.at[0], is0)
            pltpu.make_async_copy(idx_hbm.at[wid, j0 + 2], ibuf.at[0], is0).wait()
            pltpu.async_copy(g_hbm.at[ibuf.at[0, 0]], buf0, sem0)

        pltpu.make_async_copy(g_hbm.at[ibuf.at[1, 0]], buf1, sem1).wait()
        pltpu.sync_copy(buf1, acc.at[ibuf.at[1, 1]], add=True)

        @pl.when(j0 + 3 < CW)
        def _():
            pltpu.async_copy(idx_hbm.at[wid, j0 + 3], ibuf.at[1], is1)

        return 0

    lax.fori_loop(0, CW // 2, body, 0)
    if CW % 2 == 1:
        pltpu.make_async_copy(g_hbm.at[ibuf.at[0, 0]], buf0, sem0).wait()
        pltpu.sync_copy(buf0, acc.at[ibuf.at[0, 1]], add=True)

    plsc.subcore_barrier()
    pltpu.sync_copy(acc.at[pl.ds(s * TPW, TPW)], out_hbm.at[c, pl.ds(s * TPW, TPW)])


@functools.partial(
    pl.kernel,
    out_type=jax.ShapeDtypeStruct((NC, NPAD), F32),
    mesh=_mesh,
    scratch_types=[
        pltpu.VMEM((CW, K), jnp.int32),
        pltpu.VMEM((CW, K), jnp.int32),
        pltpu.VMEM((K,), F32),
        pltpu.VMEM((TPW,), F32),
        pltpu.VMEM_SHARED((NPAD,), F32),
    ],
)
def _wsum(dinv_hbm, roww_hbm, colw_hbm, out_hbm, rowv, colv, vals, zbuf, wacc):
    c = lax.axis_index("c")
    s = lax.axis_index("s")
    wid = s * NC + c
    pltpu.sync_copy(roww_hbm.at[wid], rowv)
    pltpu.sync_copy(colw_hbm.at[wid], colv)
    _zero_1d(zbuf)
    pltpu.sync_copy(zbuf, wacc.at[pl.ds(s * TPW, TPW)])
    plsc.subcore_barrier()

    def body(j, _):
        pltpu.sync_copy(dinv_hbm.at[colv.at[j]], vals)
        pltpu.sync_copy(vals, wacc.at[rowv.at[j]], add=True)
        return 0

    lax.fori_loop(0, CW, body, 0)
    plsc.subcore_barrier()
    pltpu.sync_copy(wacc.at[pl.ds(s * TPW, TPW)], out_hbm.at[c, pl.ds(s * TPW, TPW)])


TB = 1000


def _tc1_body(x_ref, w_ref, d0_ref, d1_ref, g_ref, dinv_ref):
    deg = d0_ref[...] + d1_ref[...] + 1.0
    dinv = lax.rsqrt(deg)
    h = jnp.dot(x_ref[...], w_ref[...], preferred_element_type=F32)
    g_ref[...] = h * dinv
    dinv_ref[...] = dinv


def _tc2_body(g_in_ref, pa_ref, pb_ref, d0_ref, d1_ref, b_ref, w_ref, out_ref):
    deg = d0_ref[...] + d1_ref[...] + 1.0
    dinv = lax.rsqrt(deg)
    su = pa_ref[...] + pb_ref[...] + g_in_ref[...]
    h = jnp.maximum(dinv * su + b_ref[...], 0.0)
    out_ref[...] = dinv * jnp.dot(h, w_ref[...], preferred_element_type=F32)


def _tc3_body(g_in_ref, pa_ref, pb_ref, d0_ref, d1_ref, w0_ref, w1_ref,
              b2_ref, w3_ref, b3_ref, out_ref, acc):
    i = pl.program_id(0)

    @pl.when(i == 0)
    def _():
        acc[...] = jnp.zeros((8, D), F32)

    deg = d0_ref[...] + d1_ref[...] + 1.0
    dinv = lax.rsqrt(deg)
    su = pa_ref[...] + pb_ref[...] + g_in_ref[...]
    h3 = jnp.maximum(dinv * su + b2_ref[...], 0.0)
    v = dinv * (w0_ref[...] + w1_ref[...] + dinv)
    acc[0:1, :] = acc[0:1, :] + jnp.sum(v * h3, axis=0, keepdims=True)

    @pl.when(i == pl.num_programs(0) - 1)
    def _():
        out_ref[...] = (
            jnp.dot(acc[0:1, :] / float(N), w3_ref[...],
                    preferred_element_type=F32) + b3_ref[...]
        )


def _row_spec(block):
    return pl.BlockSpec(block, lambda i: (i, 0))


def _const_spec(block):
    return pl.BlockSpec(block, lambda i: (0, 0))


_tc1 = pl.pallas_call(
    _tc1_body,
    grid=(N // TB,),
    in_specs=[_row_spec((TB, D)), _const_spec((D, D)),
              _row_spec((TB, 1)), _row_spec((TB, 1))],
    out_specs=[_row_spec((TB, D)), _row_spec((TB, 1))],
    out_shape=[jax.ShapeDtypeStruct((N, D), F32),
               jax.ShapeDtypeStruct((N, 1), F32)],
)

_tc2 = pl.pallas_call(
    _tc2_body,
    grid=(N // TB,),
    in_specs=[_row_spec((TB, D)), _row_spec((TB, D)), _row_spec((TB, D)),
              _row_spec((TB, 1)), _row_spec((TB, 1)),
              _const_spec((1, D)), _const_spec((D, D))],
    out_specs=_row_spec((TB, D)),
    out_shape=jax.ShapeDtypeStruct((N, D), F32),
)

_tc3 = pl.pallas_call(
    _tc3_body,
    grid=(N // TB,),
    in_specs=[_row_spec((TB, D)), _row_spec((TB, D)), _row_spec((TB, D)),
              _row_spec((TB, 1)), _row_spec((TB, 1)),
              _row_spec((TB, 1)), _row_spec((TB, 1)),
              _const_spec((1, D)), _const_spec((D, D)), _const_spec((1, D))],
    out_specs=_const_spec((1, D)),
    out_shape=jax.ShapeDtypeStruct((1, D), F32),
    scratch_shapes=[pltpu.VMEM((8, D), F32)],
)


def kernel(x, edge_index, W1, b1, W2, b2, W3, b3):
    row = edge_index[0]
    col = edge_index[1]

    pad = NW * EPW - E
    pr = jnp.arange(pad, dtype=jnp.int32)
    roww = jnp.concatenate([row, pr % N]).reshape(NW, CW, K)
    colw = jnp.concatenate([col, N + pr % (NPAD - N)]).reshape(NW, CW, K)
    idxp = jnp.stack([roww, colw], axis=2)

    degp = _deg(colw)
    d0 = degp[0, :N, None]
    d1 = degp[1, :N, None]

    g1, dinv = _tc1(x, W1, d0, d1)
    dinv_pad = jnp.concatenate([dinv[:, 0], jnp.zeros((NPAD - N,), F32)])

    p1 = _agg(g1, idxp)
    g2 = _tc2(g1, p1[0, :N], p1[1, :N], d0, d1, b1.reshape(1, D), W2)
    p2 = _agg(g2, idxp)
    wp = _wsum(dinv_pad, roww, colw)

    return _tc3(g2, p2[0, :N], p2[1, :N], d0, d1,
                wp[0, :N, None], wp[1, :N, None],
                b2.reshape(1, D), W3, b3.reshape(1, D))


# speedup vs baseline: 33.7409x; 1.1091x over previous
import functools

import jax
import jax.numpy as jnp
from jax import lax
from jax.experimental import pallas as pl
from jax.experimental.pallas import tpu as pltpu
from jax.experimental.pallas import tpu_sc as plsc

F32 = jnp.float32

N = 10000
E = 320000
D = 128

NC = 2
NS = 16
NW = NC * NS
K = 112
CW = -(-E // (NW * K))
EPW = CW * K
NPAD = 10240
NPAD2 = 10112
TPW = NPAD // NS
TPW2 = NPAD2 // NS

_mesh = plsc.VectorSubcoreMesh(core_axis_name="c", subcore_axis_name="s")


def _zero_rows(buf):
    z = jnp.zeros((16,), F32)

    def body(i, _):
        for k in range(buf.shape[1] // 16):
            buf[i, pl.ds(k * 16, 16)] = z
        return 0

    lax.fori_loop(0, buf.shape[0], body, 0)


def _zero_1d(buf):
    z = jnp.zeros((16,), F32)

    def body(i, _):
        buf[pl.ds(i * 16, 16)] = z
        return 0

    lax.fori_loop(0, buf.shape[0] // 16, body, 0)


@functools.partial(
    pl.kernel,
    out_type=jax.ShapeDtypeStruct((NC, NPAD), F32),
    mesh=_mesh,
    scratch_types=[
        pltpu.VMEM((CW, K), jnp.int32),
        pltpu.VMEM((K,), F32),
        pltpu.VMEM((TPW,), F32),
        pltpu.VMEM_SHARED((NPAD,), F32),
    ],
)
def _deg(colw_hbm, out_hbm, colv, ones, zbuf, dacc):
    c = lax.axis_index("c")
    s = lax.axis_index("s")
    wid = s * NC + c
    pltpu.sync_copy(colw_hbm.at[wid], colv)
    for k in range(K // 16):
        ones[pl.ds(k * 16, 16)] = jnp.ones((16,), F32)
    _zero_1d(zbuf)
    pltpu.sync_copy(zbuf, dacc.at[pl.ds(s * TPW, TPW)])
    plsc.subcore_barrier()

    def body(j, _):
        pltpu.sync_copy(ones, dacc.at[colv.at[j]], add=True)
        return 0

    lax.fori_loop(0, CW, body, 0)
    plsc.subcore_barrier()
    pltpu.sync_copy(dacc.at[pl.ds(s * TPW, TPW)], out_hbm.at[c, pl.ds(s * TPW, TPW)])


def _agg_body(with_wsum, g_hbm, idx_hbm, *args):
    if with_wsum:
        (dinv_hbm, out_hbm, wout_hbm, ibuf, b0, b1, b2, acc, dsp, wacc, vals,
         zbuf, s0, s1, s2, i0, i1, i2) = args
    else:
        (out_hbm, ibuf, b0, b1, b2, acc, s0, s1, s2, i0, i1, i2) = args
    bufs = (b0, b1, b2)
    gsem = (s0, s1, s2)
    isem = (i0, i1, i2)
    c = lax.axis_index("c")
    s = lax.axis_index("s")
    wid = s * NC + c

    _zero_rows(b0)
    nfull = TPW2 // K
    for b in range(nfull):
        pltpu.sync_copy(b0, acc.at[pl.ds(s * TPW2 + b * K, K)])
    rem = TPW2 - nfull * K
    if rem:
        pltpu.sync_copy(b0.at[pl.ds(0, rem)],
                        acc.at[pl.ds(s * TPW2 + nfull * K, rem)])
    if with_wsum:
        _zero_1d(zbuf)
        pltpu.sync_copy(zbuf, wacc.at[pl.ds(s * TPW, TPW)])

        @pl.when(s == 0)
        def _():
            pltpu.sync_copy(dinv_hbm, dsp)

    plsc.subcore_barrier()

    def idx_cp(j, t):
        return pltpu.make_async_copy(idx_hbm.at[wid, j], ibuf.at[t], isem[t])

    def g_cp(j, t):
        return pltpu.make_async_copy(g_hbm.at[ibuf.at[t, 0]], bufs[t], gsem[t])

    idx_cp(0, 0).start()
    idx_cp(1, 1).start()
    idx_cp(2, 2).start()
    idx_cp(0, 0).wait()
    g_cp(0, 0).start()
    idx_cp(1, 1).wait()
    g_cp(1, 1).start()

    def body(k, _):
        j0 = 3 * k
        for t in range(3):
            j = j0 + t
            tn = (t + 2) % 3

            @pl.when(j + 2 < CW)
            def _():
                idx_cp(j + 2, tn).wait()
                g_cp(j + 2, tn).start()

            g_cp(j, t).wait()
            pltpu.sync_copy(bufs[t], acc.at[ibuf.at[t, 1]], add=True)
            if with_wsum:
                pltpu.sync_copy(dsp.at[ibuf.at[t, 1]], vals)
                pltpu.sync_copy(vals, wacc.at[ibuf.at[t, 0]], add=True)

            @pl.when(j + 3 < CW)
            def _():
                idx_cp(j + 3, t).start()
        return 0

    lax.fori_loop(0, CW // 3, body, 0)

    plsc.subcore_barrier()
    pltpu.sync_copy(acc.at[pl.ds(s * TPW2, TPW2)],
                    out_hbm.at[c, pl.ds(s * TPW2, TPW2)])
    if with_wsum:
        pltpu.sync_copy(wacc.at[pl.ds(s * TPW, TPW)],
                        wout_hbm.at[c, pl.ds(s * TPW, TPW)])


_agg = functools.partial(
    pl.kernel,
    out_type=jax.ShapeDtypeStruct((NC, NPAD2, D), F32),
    mesh=_mesh,
    scratch_types=[
        pltpu.VMEM((3, 2, K), jnp.int32),
        pltpu.VMEM((K, D), F32),
        pltpu.VMEM((K, D), F32),
        pltpu.VMEM((K, D), F32),
        pltpu.VMEM_SHARED((NPAD2, D), F32),
        pltpu.SemaphoreType.DMA,
        pltpu.SemaphoreType.DMA,
        pltpu.SemaphoreType.DMA,
        pltpu.SemaphoreType.DMA,
        pltpu.SemaphoreType.DMA,
        pltpu.SemaphoreType.DMA,
    ],
)(functools.partial(_agg_body, False))

_aggw = functools.partial(
    pl.kernel,
    out_type=[jax.ShapeDtypeStruct((NC, NPAD2, D), F32),
              jax.ShapeDtypeStruct((NC, NPAD), F32)],
    mesh=_mesh,
    scratch_types=[
        pltpu.VMEM((3, 2, K), jnp.int32),
        pltpu.VMEM((K, D), F32),
        pltpu.VMEM((K, D), F32),
        pltpu.VMEM((K, D), F32),
        pltpu.VMEM_SHARED((NPAD2, D), F32),
        pltpu.VMEM_SHARED((NPAD,), F32),
        pltpu.VMEM_SHARED((NPAD,), F32),
        pltpu.VMEM((K,), F32),
        pltpu.VMEM((TPW,), F32),
        pltpu.SemaphoreType.DMA,
        pltpu.SemaphoreType.DMA,
        pltpu.SemaphoreType.DMA,
        pltpu.SemaphoreType.DMA,
        pltpu.SemaphoreType.DMA,
        pltpu.SemaphoreType.DMA,
    ],
)(functools.partial(_agg_body, True))


TB = 1000


def _tc1_body(x_ref, w_ref, d0_ref, d1_ref, g_ref, dinv_ref):
    deg = d0_ref[...] + d1_ref[...] + 1.0
    dinv = lax.rsqrt(deg)
    h = jnp.dot(x_ref[...], w_ref[...], preferred_element_type=F32)
    g_ref[...] = h * dinv
    dinv_ref[...] = dinv


def _tc2_body(g_in_ref, pa_ref, pb_ref, d0_ref, d1_ref, b_ref, w_ref, out_ref):
    deg = d0_ref[...] + d1_ref[...] + 1.0
    dinv = lax.rsqrt(deg)
    su = pa_ref[...] + pb_ref[...] + g_in_ref[...]
    h = jnp.maximum(dinv * su + b_ref[...], 0.0)
    out_ref[...] = dinv * jnp.dot(h, w_ref[...], preferred_element_type=F32)


def _tc3_body(g_in_ref, pa_ref, pb_ref, d0_ref, d1_ref, w0_ref, w1_ref,
              b2_ref, w3_ref, b3_ref, out_ref, acc):
    i = pl.program_id(0)

    @pl.when(i == 0)
    def _():
        acc[...] = jnp.zeros((8, D), F32)

    deg = d0_ref[...] + d1_ref[...] + 1.0
    dinv = lax.rsqrt(deg)
    su = pa_ref[...] + pb_ref[...] + g_in_ref[...]
    h3 = jnp.maximum(dinv * su + b2_ref[...], 0.0)
    v = dinv * (w0_ref[...] + w1_ref[...] + dinv)
    acc[0:1, :] = acc[0:1, :] + jnp.sum(v * h3, axis=0, keepdims=True)

    @pl.when(i == pl.num_programs(0) - 1)
    def _():
        out_ref[...] = (
            jnp.dot(acc[0:1, :] / float(N), w3_ref[...],
                    preferred_element_type=F32) + b3_ref[...]
        )


def _row_spec(block):
    return pl.BlockSpec(block, lambda i: (i, 0))


def _const_spec(block):
    return pl.BlockSpec(block, lambda i: (0, 0))


_tc1 = pl.pallas_call(
    _tc1_body,
    grid=(N // TB,),
    in_specs=[_row_spec((TB, D)), _const_spec((D, D)),
              _row_spec((TB, 1)), _row_spec((TB, 1))],
    out_specs=[_row_spec((TB, D)), _row_spec((TB, 1))],
    out_shape=[jax.ShapeDtypeStruct((N, D), F32),
               jax.ShapeDtypeStruct((N, 1), F32)],
)

_tc2 = pl.pallas_call(
    _tc2_body,
    grid=(N // TB,),
    in_specs=[_row_spec((TB, D)), _row_spec((TB, D)), _row_spec((TB, D)),
              _row_spec((TB, 1)), _row_spec((TB, 1)),
              _const_spec((1, D)), _const_spec((D, D))],
    out_specs=_row_spec((TB, D)),
    out_shape=jax.ShapeDtypeStruct((N, D), F32),
)

_tc3 = pl.pallas_call(
    _tc3_body,
    grid=(N // TB,),
    in_specs=[_row_spec((TB, D)), _row_spec((TB, D)), _row_spec((TB, D)),
              _row_spec((TB, 1)), _row_spec((TB, 1)),
              _row_spec((TB, 1)), _row_spec((TB, 1)),
              _const_spec((1, D)), _const_spec((D, D)), _const_spec((1, D))],
    out_specs=_const_spec((1, D)),
    out_shape=jax.ShapeDtypeStruct((1, D), F32),
    scratch_shapes=[pltpu.VMEM((8, D), F32)],
)


def kernel(x, edge_index, W1, b1, W2, b2, W3, b3):
    row = edge_index[0]
    col = edge_index[1]

    pad = NW * EPW - E
    pr = jnp.arange(pad, dtype=jnp.int32)
    roww = jnp.concatenate([row, pr % N]).reshape(NW, CW, K)
    colw = jnp.concatenate([col, N + pr % (NPAD2 - N)]).reshape(NW, CW, K)
    idxp = jnp.stack([roww, colw], axis=2)

    degp = _deg(colw)
    d0 = degp[0, :N, None]
    d1 = degp[1, :N, None]

    g1, dinv = _tc1(x, W1, d0, d1)
    dinv_pad = jnp.concatenate([dinv[:, 0], jnp.zeros((NPAD - N,), F32)])

    p1 = _agg(g1, idxp)
    g2 = _tc2(g1, p1[0, :N], p1[1, :N], d0, d1, b1.reshape(1, D), W2)
    p2, wp = _aggw(g2, idxp, dinv_pad)

    return _tc3(g2, p2[0, :N], p2[1, :N], d0, d1,
                wp[0, :N, None], wp[1, :N, None],
                b2.reshape(1, D), W3, b3.reshape(1, D))


# speedup vs baseline: 39.8068x; 1.1798x over previous
import functools

import jax
import jax.numpy as jnp
from jax import lax
from jax.experimental import pallas as pl
from jax.experimental.pallas import tpu as pltpu
from jax.experimental.pallas import tpu_sc as plsc

F32 = jnp.float32

N = 10000
E = 320000
D = 128

NC = 2
NS = 16
NW = NC * NS
K = 112
CW = -(-E // (NW * K))
EPW = CW * K
NPAD = 10240
NPAD2 = 10112
TPW = NPAD // NS
TPW2 = NPAD2 // NS

_mesh = plsc.VectorSubcoreMesh(core_axis_name="c", subcore_axis_name="s")


def _zero_rows(buf):
    z = jnp.zeros((16,), F32)

    def body(i, _):
        for k in range(buf.shape[1] // 16):
            buf[i, pl.ds(k * 16, 16)] = z
        return 0

    lax.fori_loop(0, buf.shape[0], body, 0)


def _zero_1d(buf):
    z = jnp.zeros((16,), F32)

    def body(i, _):
        buf[pl.ds(i * 16, 16)] = z
        return 0

    lax.fori_loop(0, buf.shape[0] // 16, body, 0)


@functools.partial(
    pl.kernel,
    out_type=jax.ShapeDtypeStruct((NC, NPAD), F32),
    mesh=_mesh,
    scratch_types=[
        pltpu.VMEM((CW, K), jnp.int32),
        pltpu.VMEM((K,), F32),
        pltpu.VMEM((TPW,), F32),
        pltpu.VMEM_SHARED((NPAD,), F32),
    ],
)
def _deg(colw_hbm, out_hbm, colv, ones, zbuf, dacc):
    c = lax.axis_index("c")
    s = lax.axis_index("s")
    wid = s * NC + c
    pltpu.sync_copy(colw_hbm.at[wid], colv)
    for k in range(K // 16):
        ones[pl.ds(k * 16, 16)] = jnp.ones((16,), F32)
    _zero_1d(zbuf)
    pltpu.sync_copy(zbuf, dacc.at[pl.ds(s * TPW, TPW)])
    plsc.subcore_barrier()

    def body(j, _):
        pltpu.sync_copy(ones, dacc.at[colv.at[j]], add=True)
        return 0

    lax.fori_loop(0, CW, body, 0)
    plsc.subcore_barrier()
    pltpu.sync_copy(dacc.at[pl.ds(s * TPW, TPW)], out_hbm.at[c, pl.ds(s * TPW, TPW)])


def _agg_body(with_wsum, g_hbm, idx_hbm, *args):
    if with_wsum:
        (dinv_hbm, out_hbm, wout_hbm, ibuf, b0, b1, b2, acc, dsp, wacc, vals,
         zbuf, s0, s1, s2, i0, i1, i2, c0, c1, c2) = args
    else:
        (out_hbm, ibuf, b0, b1, b2, acc, s0, s1, s2, i0, i1, i2,
         c0, c1, c2) = args
    bufs = (b0, b1, b2)
    gsem = (s0, s1, s2)
    isem = (i0, i1, i2)
    ssem = (c0, c1, c2)
    c = lax.axis_index("c")
    s = lax.axis_index("s")
    wid = s * NC + c

    _zero_rows(b0)
    nfull = TPW2 // K
    for b in range(nfull):
        pltpu.sync_copy(b0, acc.at[pl.ds(s * TPW2 + b * K, K)])
    rem = TPW2 - nfull * K
    if rem:
        pltpu.sync_copy(b0.at[pl.ds(0, rem)],
                        acc.at[pl.ds(s * TPW2 + nfull * K, rem)])
    if with_wsum:
        _zero_1d(zbuf)
        pltpu.sync_copy(zbuf, wacc.at[pl.ds(s * TPW, TPW)])

        @pl.when(s == 0)
        def _():
            pltpu.sync_copy(dinv_hbm, dsp)

    plsc.subcore_barrier()

    def idx_cp(j, t):
        return pltpu.make_async_copy(idx_hbm.at[wid, j], ibuf.at[t], isem[t])

    def g_cp(j, t):
        return pltpu.make_async_copy(g_hbm.at[ibuf.at[t, 0]], bufs[t], gsem[t])

    def sc_start(t):
        pltpu.async_copy(bufs[t], acc.at[ibuf.at[t, 1]], ssem[t], add=True)

    def sc_wait(t):
        pltpu.make_async_copy(bufs[t], acc.at[ibuf.at[t, 1]], ssem[t]).wait()

    idx_cp(0, 0).start()
    idx_cp(1, 1).start()
    idx_cp(2, 2).start()
    idx_cp(0, 0).wait()
    g_cp(0, 0).start()
    idx_cp(1, 1).wait()
    g_cp(1, 1).start()

    def body(k, _):
        j0 = 3 * k
        for t in range(3):
            j = j0 + t
            tn = (t + 2) % 3

            @pl.when(j + 2 < CW)
            def _():
                idx_cp(j + 2, tn).wait()

                @pl.when(j >= 1)
                def _():
                    sc_wait(tn)

                g_cp(j + 2, tn).start()

            g_cp(j, t).wait()
            if with_wsum:
                pltpu.sync_copy(dsp.at[ibuf.at[t, 1]], vals)
                pltpu.sync_copy(vals, wacc.at[ibuf.at[t, 0]], add=True)
            sc_start(t)

            @pl.when(j + 3 < CW)
            def _():
                idx_cp(j + 3, t).start()
        return 0

    lax.fori_loop(0, CW // 3, body, 0)
    for t in range(3):
        sc_wait(t)

    plsc.subcore_barrier()
    pltpu.sync_copy(acc.at[pl.ds(s * TPW2, TPW2)],
                    out_hbm.at[c, pl.ds(s * TPW2, TPW2)])
    if with_wsum:
        pltpu.sync_copy(wacc.at[pl.ds(s * TPW, TPW)],
                        wout_hbm.at[c, pl.ds(s * TPW, TPW)])


_agg = functools.partial(
    pl.kernel,
    out_type=jax.ShapeDtypeStruct((NC, NPAD2, D), F32),
    mesh=_mesh,
    scratch_types=[
        pltpu.VMEM((3, 2, K), jnp.int32),
        pltpu.VMEM((K, D), F32),
        pltpu.VMEM((K, D), F32),
        pltpu.VMEM((K, D), F32),
        pltpu.VMEM_SHARED((NPAD2, D), F32),
        pltpu.SemaphoreType.DMA,
        pltpu.SemaphoreType.DMA,
        pltpu.SemaphoreType.DMA,
        pltpu.SemaphoreType.DMA,
        pltpu.SemaphoreType.DMA,
        pltpu.SemaphoreType.DMA,
        pltpu.SemaphoreType.DMA,
        pltpu.SemaphoreType.DMA,
        pltpu.SemaphoreType.DMA,
    ],
)(functools.partial(_agg_body, False))

_aggw = functools.partial(
    pl.kernel,
    out_type=[jax.ShapeDtypeStruct((NC, NPAD2, D), F32),
              jax.ShapeDtypeStruct((NC, NPAD), F32)],
    mesh=_mesh,
    scratch_types=[
        pltpu.VMEM((3, 2, K), jnp.int32),
        pltpu.VMEM((K, D), F32),
        pltpu.VMEM((K, D), F32),
        pltpu.VMEM((K, D), F32),
        pltpu.VMEM_SHARED((NPAD2, D), F32),
        pltpu.VMEM_SHARED((NPAD,), F32),
        pltpu.VMEM_SHARED((NPAD,), F32),
        pltpu.VMEM((K,), F32),
        pltpu.VMEM((TPW,), F32),
        pltpu.SemaphoreType.DMA,
        pltpu.SemaphoreType.DMA,
        pltpu.SemaphoreType.DMA,
        pltpu.SemaphoreType.DMA,
        pltpu.SemaphoreType.DMA,
        pltpu.SemaphoreType.DMA,
        pltpu.SemaphoreType.DMA,
        pltpu.SemaphoreType.DMA,
        pltpu.SemaphoreType.DMA,
    ],
)(functools.partial(_agg_body, True))


TB = 1000


def _tc1_body(x_ref, w_ref, d0_ref, d1_ref, g_ref, dinv_ref):
    deg = d0_ref[...] + d1_ref[...] + 1.0
    dinv = lax.rsqrt(deg)
    h = jnp.dot(x_ref[...], w_ref[...], preferred_element_type=F32)
    g_ref[...] = h * dinv
    dinv_ref[...] = dinv


def _tc2_body(g_in_ref, pa_ref, pb_ref, d0_ref, d1_ref, b_ref, w_ref, out_ref):
    deg = d0_ref[...] + d1_ref[...] + 1.0
    dinv = lax.rsqrt(deg)
    su = pa_ref[...] + pb_ref[...] + g_in_ref[...]
    h = jnp.maximum(dinv * su + b_ref[...], 0.0)
    out_ref[...] = dinv * jnp.dot(h, w_ref[...], preferred_element_type=F32)


def _tc3_body(g_in_ref, pa_ref, pb_ref, d0_ref, d1_ref, w0_ref, w1_ref,
              b2_ref, w3_ref, b3_ref, out_ref, acc):
    i = pl.program_id(0)

    @pl.when(i == 0)
    def _():
        acc[...] = jnp.zeros((8, D), F32)

    deg = d0_ref[...] + d1_ref[...] + 1.0
    dinv = lax.rsqrt(deg)
    su = pa_ref[...] + pb_ref[...] + g_in_ref[...]
    h3 = jnp.maximum(dinv * su + b2_ref[...], 0.0)
    v = dinv * (w0_ref[...] + w1_ref[...] + dinv)
    acc[0:1, :] = acc[0:1, :] + jnp.sum(v * h3, axis=0, keepdims=True)

    @pl.when(i == pl.num_programs(0) - 1)
    def _():
        out_ref[...] = (
            jnp.dot(acc[0:1, :] / float(N), w3_ref[...],
                    preferred_element_type=F32) + b3_ref[...]
        )


def _row_spec(block):
    return pl.BlockSpec(block, lambda i: (i, 0))


def _const_spec(block):
    return pl.BlockSpec(block, lambda i: (0, 0))


_tc1 = pl.pallas_call(
    _tc1_body,
    grid=(N // TB,),
    in_specs=[_row_spec((TB, D)), _const_spec((D, D)),
              _row_spec((TB, 1)), _row_spec((TB, 1))],
    out_specs=[_row_spec((TB, D)), _row_spec((TB, 1))],
    out_shape=[jax.ShapeDtypeStruct((N, D), F32),
               jax.ShapeDtypeStruct((N, 1), F32)],
)

_tc2 = pl.pallas_call(
    _tc2_body,
    grid=(N // TB,),
    in_specs=[_row_spec((TB, D)), _row_spec((TB, D)), _row_spec((TB, D)),
              _row_spec((TB, 1)), _row_spec((TB, 1)),
              _const_spec((1, D)), _const_spec((D, D))],
    out_specs=_row_spec((TB, D)),
    out_shape=jax.ShapeDtypeStruct((N, D), F32),
)

_tc3 = pl.pallas_call(
    _tc3_body,
    grid=(N // TB,),
    in_specs=[_row_spec((TB, D)), _row_spec((TB, D)), _row_spec((TB, D)),
              _row_spec((TB, 1)), _row_spec((TB, 1)),
              _row_spec((TB, 1)), _row_spec((TB, 1)),
              _const_spec((1, D)), _const_spec((D, D)), _const_spec((1, D))],
    out_specs=_const_spec((1, D)),
    out_shape=jax.ShapeDtypeStruct((1, D), F32),
    scratch_shapes=[pltpu.VMEM((8, D), F32)],
)


def kernel(x, edge_index, W1, b1, W2, b2, W3, b3):
    row = edge_index[0]
    col = edge_index[1]

    pad = NW * EPW - E
    pr = jnp.arange(pad, dtype=jnp.int32)
    roww = jnp.concatenate([row, pr % N]).reshape(NW, CW, K)
    colw = jnp.concatenate([col, N + pr % (NPAD2 - N)]).reshape(NW, CW, K)
    idxp = jnp.stack([roww, colw], axis=2)

    degp = _deg(colw)
    d0 = degp[0, :N, None]
    d1 = degp[1, :N, None]

    g1, dinv = _tc1(x, W1, d0, d1)
    dinv_pad = jnp.concatenate([dinv[:, 0], jnp.zeros((NPAD - N,), F32)])

    p1 = _agg(g1, idxp)
    g2 = _tc2(g1, p1[0, :N], p1[1, :N], d0, d1, b1.reshape(1, D), W2)
    p2, wp = _aggw(g2, idxp, dinv_pad)

    return _tc3(g2, p2[0, :N], p2[1, :N], d0, d1,
                wp[0, :N, None], wp[1, :N, None],
                b2.reshape(1, D), W3, b3.reshape(1, D))


# speedup vs baseline: 41.8004x; 1.0501x over previous
import functools

import jax
import jax.numpy as jnp
from jax import lax
from jax.experimental import pallas as pl
from jax.experimental.pallas import tpu as pltpu
from jax.experimental.pallas import tpu_sc as plsc

F32 = jnp.float32

N = 10000
E = 320000
D = 128

NC = 2
NS = 16
NW = NC * NS
K = 112
CW = -(-E // (NW * K))
EPW = CW * K
NPAD = 10240
NPAD2 = 10112
TPW = NPAD // NS
TPW2 = NPAD2 // NS

_mesh = plsc.VectorSubcoreMesh(core_axis_name="c", subcore_axis_name="s")


def _zero_rows(buf):
    z = jnp.zeros((16,), F32)

    def body(i, _):
        for k in range(buf.shape[1] // 16):
            buf[i, pl.ds(k * 16, 16)] = z
        return 0

    lax.fori_loop(0, buf.shape[0], body, 0)


def _zero_1d(buf):
    z = jnp.zeros((16,), F32)

    def body(i, _):
        buf[pl.ds(i * 16, 16)] = z
        return 0

    lax.fori_loop(0, buf.shape[0] // 16, body, 0)


@functools.partial(
    pl.kernel,
    out_type=jax.ShapeDtypeStruct((NC, NPAD), F32),
    mesh=_mesh,
    scratch_types=[
        pltpu.VMEM((CW, K), jnp.int32),
        pltpu.VMEM((K,), F32),
        pltpu.VMEM((TPW,), F32),
        pltpu.VMEM_SHARED((NPAD,), F32),
    ],
)
def _deg(colw_hbm, out_hbm, colv, ones, zbuf, dacc):
    c = lax.axis_index("c")
    s = lax.axis_index("s")
    wid = s * NC + c
    pltpu.sync_copy(colw_hbm.at[wid], colv)
    for k in range(K // 16):
        ones[pl.ds(k * 16, 16)] = jnp.ones((16,), F32)
    _zero_1d(zbuf)
    pltpu.sync_copy(zbuf, dacc.at[pl.ds(s * TPW, TPW)])
    plsc.subcore_barrier()

    def body(j, _):
        pltpu.sync_copy(ones, dacc.at[colv.at[j]], add=True)
        return 0

    lax.fori_loop(0, CW, body, 0)
    plsc.subcore_barrier()
    pltpu.sync_copy(dacc.at[pl.ds(s * TPW, TPW)], out_hbm.at[c, pl.ds(s * TPW, TPW)])


def _agg_body(with_wsum, g_hbm, idx_hbm, *args):
    if with_wsum:
        (dinv_hbm, out_hbm, wout_hbm, ibuf, b0, b1, b2, acc, dsp, wacc, vals,
         zbuf, s0, s1, s2, i0, i1, i2, c0, c1, c2) = args
    else:
        (out_hbm, ibuf, b0, b1, b2, acc, s0, s1, s2, i0, i1, i2,
         c0, c1, c2) = args
    bufs = (b0, b1, b2)
    gsem = (s0, s1, s2)
    isem = (i0, i1, i2)
    ssem = (c0, c1, c2)
    c = lax.axis_index("c")
    s = lax.axis_index("s")
    wid = s * NC + c

    _zero_rows(b0)
    nfull = TPW2 // K
    for b in range(nfull):
        pltpu.sync_copy(b0, acc.at[pl.ds(s * TPW2 + b * K, K)])
    rem = TPW2 - nfull * K
    if rem:
        pltpu.sync_copy(b0.at[pl.ds(0, rem)],
                        acc.at[pl.ds(s * TPW2 + nfull * K, rem)])
    if with_wsum:
        _zero_1d(zbuf)
        pltpu.sync_copy(zbuf, wacc.at[pl.ds(s * TPW, TPW)])

        @pl.when(s == 0)
        def _():
            pltpu.sync_copy(dinv_hbm, dsp)

    plsc.subcore_barrier()

    def idx_cp(j, t):
        return pltpu.make_async_copy(idx_hbm.at[wid, j], ibuf.at[t], isem[t])

    def g_cp(j, t):
        return pltpu.make_async_copy(g_hbm.at[ibuf.at[t, 0]], bufs[t], gsem[t])

    def sc_start(t):
        pltpu.async_copy(bufs[t], acc.at[ibuf.at[t, 1]], ssem[t], add=True)

    def sc_wait(t):
        pltpu.make_async_copy(bufs[t], acc.at[ibuf.at[t, 1]], ssem[t]).wait()

    idx_cp(0, 0).start()
    idx_cp(1, 1).start()
    idx_cp(2, 2).start()
    idx_cp(0, 0).wait()
    g_cp(0, 0).start()
    idx_cp(1, 1).wait()
    g_cp(1, 1).start()

    def body(k, _):
        j0 = 3 * k
        for t in range(3):
            j = j0 + t
            tn = (t + 2) % 3

            @pl.when(j + 2 < CW)
            def _():
                idx_cp(j + 2, tn).wait()

                @pl.when(j >= 1)
                def _():
                    sc_wait(tn)

                g_cp(j + 2, tn).start()

            g_cp(j, t).wait()
            if with_wsum:
                pltpu.sync_copy(dsp.at[ibuf.at[t, 1]], vals)
                pltpu.sync_copy(vals, wacc.at[ibuf.at[t, 0]], add=True)
            sc_start(t)

            @pl.when(j + 3 < CW)
            def _():
                idx_cp(j + 3, t).start()
        return 0

    lax.fori_loop(0, CW // 3, body, 0)
    for t in range(3):
        sc_wait(t)

    plsc.subcore_barrier()
    pltpu.sync_copy(acc.at[pl.ds(s * TPW2, TPW2)],
                    out_hbm.at[c, pl.ds(s * TPW2, TPW2)])
    if with_wsum:
        pltpu.sync_copy(wacc.at[pl.ds(s * TPW, TPW)],
                        wout_hbm.at[c, pl.ds(s * TPW, TPW)])


_agg = functools.partial(
    pl.kernel,
    out_type=jax.ShapeDtypeStruct((NC, NPAD2, D), F32),
    mesh=_mesh,
    scratch_types=[
        pltpu.VMEM((3, 2, K), jnp.int32),
        pltpu.VMEM((K, D), F32),
        pltpu.VMEM((K, D), F32),
        pltpu.VMEM((K, D), F32),
        pltpu.VMEM_SHARED((NPAD2, D), F32),
        pltpu.SemaphoreType.DMA,
        pltpu.SemaphoreType.DMA,
        pltpu.SemaphoreType.DMA,
        pltpu.SemaphoreType.DMA,
        pltpu.SemaphoreType.DMA,
        pltpu.SemaphoreType.DMA,
        pltpu.SemaphoreType.DMA,
        pltpu.SemaphoreType.DMA,
        pltpu.SemaphoreType.DMA,
    ],
)(functools.partial(_agg_body, False))

_aggw = functools.partial(
    pl.kernel,
    out_type=[jax.ShapeDtypeStruct((NC, NPAD2, D), F32),
              jax.ShapeDtypeStruct((NC, NPAD), F32)],
    mesh=_mesh,
    scratch_types=[
        pltpu.VMEM((3, 2, K), jnp.int32),
        pltpu.VMEM((K, D), F32),
        pltpu.VMEM((K, D), F32),
        pltpu.VMEM((K, D), F32),
        pltpu.VMEM_SHARED((NPAD2, D), F32),
        pltpu.VMEM_SHARED((NPAD,), F32),
        pltpu.VMEM_SHARED((NPAD,), F32),
        pltpu.VMEM((K,), F32),
        pltpu.VMEM((TPW,), F32),
        pltpu.SemaphoreType.DMA,
        pltpu.SemaphoreType.DMA,
        pltpu.SemaphoreType.DMA,
        pltpu.SemaphoreType.DMA,
        pltpu.SemaphoreType.DMA,
        pltpu.SemaphoreType.DMA,
        pltpu.SemaphoreType.DMA,
        pltpu.SemaphoreType.DMA,
        pltpu.SemaphoreType.DMA,
    ],
)(functools.partial(_agg_body, True))


TB = 1000


def _dinv_from(dg_ref):
    deg = dg_ref[0] + dg_ref[1] + 1.0
    return lax.rsqrt(deg)


def _tc1_body(x_ref, w_ref, dg_ref, g_ref, dinv_ref):
    dinv = _dinv_from(dg_ref)
    h = jnp.dot(x_ref[...], w_ref[...], preferred_element_type=F32)
    g_ref[...] = h * dinv
    dinv_ref[...] = dinv


def _tc2_body(g_in_ref, p_ref, dg_ref, b_ref, w_ref, out_ref):
    dinv = _dinv_from(dg_ref)
    su = p_ref[0] + p_ref[1] + g_in_ref[...]
    h = jnp.maximum(dinv * su + b_ref[...], 0.0)
    out_ref[...] = dinv * jnp.dot(h, w_ref[...], preferred_element_type=F32)


def _tc3_body(g_in_ref, p_ref, dg_ref, wp_ref, b2_ref, w3_ref, b3_ref,
              out_ref, acc):
    i = pl.program_id(0)

    @pl.when(i == 0)
    def _():
        acc[...] = jnp.zeros((8, D), F32)

    dinv = _dinv_from(dg_ref)
    su = p_ref[0] + p_ref[1] + g_in_ref[...]
    h3 = jnp.maximum(dinv * su + b2_ref[...], 0.0)
    v = dinv * (wp_ref[0] + wp_ref[1] + dinv)
    acc[0:1, :] = acc[0:1, :] + jnp.sum(v * h3, axis=0, keepdims=True)

    @pl.when(i == pl.num_programs(0) - 1)
    def _():
        out_ref[...] = (
            jnp.dot(acc[0:1, :] / float(N), w3_ref[...],
                    preferred_element_type=F32) + b3_ref[...]
        )


def _row_spec(block):
    return pl.BlockSpec(block, lambda i: (i, 0))


def _const_spec(block):
    return pl.BlockSpec(block, lambda i: (0, 0))


_p_spec = pl.BlockSpec((2, TB, D), lambda i: (0, i, 0))
_v_spec = pl.BlockSpec((2, TB, 1), lambda i: (0, i, 0))

_tc1 = pl.pallas_call(
    _tc1_body,
    grid=(N // TB,),
    in_specs=[_row_spec((TB, D)), _const_spec((D, D)), _v_spec],
    out_specs=[_row_spec((TB, D)), _row_spec((TB, 1))],
    out_shape=[jax.ShapeDtypeStruct((N, D), F32),
               jax.ShapeDtypeStruct((N, 1), F32)],
)

_tc2 = pl.pallas_call(
    _tc2_body,
    grid=(N // TB,),
    in_specs=[_row_spec((TB, D)), _p_spec, _v_spec,
              _const_spec((1, D)), _const_spec((D, D))],
    out_specs=_row_spec((TB, D)),
    out_shape=jax.ShapeDtypeStruct((N, D), F32),
)

_tc3 = pl.pallas_call(
    _tc3_body,
    grid=(N // TB,),
    in_specs=[_row_spec((TB, D)), _p_spec, _v_spec, _v_spec,
              _const_spec((1, D)), _const_spec((D, D)), _const_spec((1, D))],
    out_specs=_const_spec((1, D)),
    out_shape=jax.ShapeDtypeStruct((1, D), F32),
    scratch_shapes=[pltpu.VMEM((8, D), F32)],
)


def kernel(x, edge_index, W1, b1, W2, b2, W3, b3):
    row = edge_index[0]
    col = edge_index[1]

    pad = NW * EPW - E
    pr = jnp.arange(pad, dtype=jnp.int32)
    roww = jnp.concatenate([row, pr % N]).reshape(NW, CW, K)
    colw = jnp.concatenate([col, N + pr % (NPAD2 - N)]).reshape(NW, CW, K)
    idxp = jnp.stack([roww, colw], axis=2)

    degp = _deg(colw).reshape(NC, NPAD, 1)

    g1, dinv = _tc1(x, W1, degp)
    dinv_pad = jnp.concatenate([dinv[:, 0], jnp.zeros((NPAD - N,), F32)])

    p1 = _agg(g1, idxp)
    g2 = _tc2(g1, p1, degp, b1.reshape(1, D), W2)
    p2, wp = _aggw(g2, idxp, dinv_pad)

    return _tc3(g2, p2, degp, wp.reshape(NC, NPAD, 1),
                b2.reshape(1, D), W3, b3.reshape(1, D))


# speedup vs baseline: 44.3710x; 1.0615x over previous
import functools

import jax
import jax.numpy as jnp
from jax import lax
from jax.experimental import pallas as pl
from jax.experimental.pallas import tpu as pltpu
from jax.experimental.pallas import tpu_sc as plsc

F32 = jnp.float32

N = 10000
E = 320000
D = 128

NC = 2
NS = 16
NW = NC * NS
K = 112
CW = -(-E // (NW * K))
EPW = CW * K
NPAD = 10240
NPAD2 = 10112
TPW = NPAD // NS
TPW2 = NPAD2 // NS

_mesh = plsc.VectorSubcoreMesh(core_axis_name="c", subcore_axis_name="s")


def _zero_rows(buf):
    z = jnp.zeros((16,), F32)

    def body(i, _):
        for k in range(buf.shape[1] // 16):
            buf[i, pl.ds(k * 16, 16)] = z
        return 0

    lax.fori_loop(0, buf.shape[0], body, 0)


def _zero_1d(buf):
    z = jnp.zeros((16,), F32)

    def body(i, _):
        buf[pl.ds(i * 16, 16)] = z
        return 0

    lax.fori_loop(0, buf.shape[0] // 16, body, 0)


@functools.partial(
    pl.kernel,
    out_type=jax.ShapeDtypeStruct((NC, NPAD), F32),
    mesh=_mesh,
    scratch_types=[
        pltpu.VMEM((CW, K), jnp.int32),
        pltpu.VMEM((K,), F32),
        pltpu.VMEM((TPW,), F32),
        pltpu.VMEM_SHARED((NPAD,), F32),
    ],
)
def _deg(colw_hbm, out_hbm, colv, ones, zbuf, dacc):
    c = lax.axis_index("c")
    s = lax.axis_index("s")
    wid = s * NC + c
    pltpu.sync_copy(colw_hbm.at[wid], colv)
    for k in range(K // 16):
        ones[pl.ds(k * 16, 16)] = jnp.ones((16,), F32)
    _zero_1d(zbuf)
    pltpu.sync_copy(zbuf, dacc.at[pl.ds(s * TPW, TPW)])
    plsc.subcore_barrier()

    def body(j, _):
        pltpu.sync_copy(ones, dacc.at[colv.at[j]], add=True)
        return 0

    lax.fori_loop(0, CW, body, 0)
    plsc.subcore_barrier()
    pltpu.sync_copy(dacc.at[pl.ds(s * TPW, TPW)], out_hbm.at[c, pl.ds(s * TPW, TPW)])


def _agg_body(with_wsum, g_hbm, idx_hbm, *args):
    if with_wsum:
        (dinv_hbm, out_hbm, wout_hbm, ibuf, b0, b1, b2, acc, dsp, wacc, vals,
         zbuf, s0, s1, s2, i0, i1, i2, i3, i4, i5, c0, c1, c2) = args
    else:
        (out_hbm, ibuf, b0, b1, b2, acc, s0, s1, s2, i0, i1, i2, i3, i4, i5,
         c0, c1, c2) = args
    bufs = (b0, b1, b2)
    gsem = (s0, s1, s2)
    isem = (i0, i1, i2, i3, i4, i5)
    ssem = (c0, c1, c2)
    c = lax.axis_index("c")
    s = lax.axis_index("s")
    wid = s * NC + c

    _zero_rows(b0)
    nfull = TPW2 // K
    for b in range(nfull):
        pltpu.sync_copy(b0, acc.at[pl.ds(s * TPW2 + b * K, K)])
    rem = TPW2 - nfull * K
    if rem:
        pltpu.sync_copy(b0.at[pl.ds(0, rem)],
                        acc.at[pl.ds(s * TPW2 + nfull * K, rem)])
    if with_wsum:
        _zero_1d(zbuf)
        pltpu.sync_copy(zbuf, wacc.at[pl.ds(s * TPW, TPW)])

        @pl.when(s == 0)
        def _():
            pltpu.sync_copy(dinv_hbm, dsp)

    plsc.subcore_barrier()

    def idx_cp(j, m):
        return pltpu.make_async_copy(idx_hbm.at[wid, j], ibuf.at[m], isem[m])

    def g_cp(t, m):
        return pltpu.make_async_copy(g_hbm.at[ibuf.at[m, 0]], bufs[t], gsem[t])

    def sc_start(t, m):
        pltpu.async_copy(bufs[t], acc.at[ibuf.at[m, 1]], ssem[t], add=True)

    def sc_wait(t, m):
        pltpu.make_async_copy(bufs[t], acc.at[ibuf.at[m, 1]], ssem[t]).wait()

    for j in range(4):
        idx_cp(j, j).start()
    idx_cp(0, 0).wait()
    g_cp(0, 0).start()
    idx_cp(1, 1).wait()
    g_cp(1, 1).start()

    def body(k, _):
        j0 = 6 * k
        for u in range(6):
            j = j0 + u
            t = u % 3
            tn = (u + 2) % 3
            mn = (u + 2) % 6

            @pl.when(j + 2 < CW)
            def _():
                idx_cp(j + 2, mn).wait()

                @pl.when(j >= 1)
                def _():
                    sc_wait(tn, (u + 5) % 6)

                g_cp(tn, mn).start()

            g_cp(t, u).wait()
            sc_start(t, u)
            if with_wsum:
                pltpu.sync_copy(dsp.at[ibuf.at[u, 1]], vals)
                pltpu.sync_copy(vals, wacc.at[ibuf.at[u, 0]], add=True)

            @pl.when(j + 4 < CW)
            def _():
                idx_cp(j + 4, (u + 4) % 6).start()
        return 0

    lax.fori_loop(0, CW // 6, body, 0)
    for j in range(CW - 3, CW):
        sc_wait(j % 3, j % 6)

    plsc.subcore_barrier()
    pltpu.sync_copy(acc.at[pl.ds(s * TPW2, TPW2)],
                    out_hbm.at[c, pl.ds(s * TPW2, TPW2)])
    if with_wsum:
        pltpu.sync_copy(wacc.at[pl.ds(s * TPW, TPW)],
                        wout_hbm.at[c, pl.ds(s * TPW, TPW)])


_agg = functools.partial(
    pl.kernel,
    out_type=jax.ShapeDtypeStruct((NC, NPAD2, D), F32),
    mesh=_mesh,
    scratch_types=[
        pltpu.VMEM((6, 2, K), jnp.int32),
        pltpu.VMEM((K, D), F32),
        pltpu.VMEM((K, D), F32),
        pltpu.VMEM((K, D), F32),
        pltpu.VMEM_SHARED((NPAD2, D), F32),
    ] + [pltpu.SemaphoreType.DMA] * 12,
)(functools.partial(_agg_body, False))

_aggw = functools.partial(
    pl.kernel,
    out_type=[jax.ShapeDtypeStruct((NC, NPAD2, D), F32),
              jax.ShapeDtypeStruct((NC, NPAD), F32)],
    mesh=_mesh,
    scratch_types=[
        pltpu.VMEM((6, 2, K), jnp.int32),
        pltpu.VMEM((K, D), F32),
        pltpu.VMEM((K, D), F32),
        pltpu.VMEM((K, D), F32),
        pltpu.VMEM_SHARED((NPAD2, D), F32),
        pltpu.VMEM_SHARED((NPAD,), F32),
        pltpu.VMEM_SHARED((NPAD,), F32),
        pltpu.VMEM((K,), F32),
        pltpu.VMEM((TPW,), F32),
    ] + [pltpu.SemaphoreType.DMA] * 12,
)(functools.partial(_agg_body, True))


TB = 1000


def _dinv_from(dg_ref):
    deg = dg_ref[0] + dg_ref[1] + 1.0
    return lax.rsqrt(deg)


def _tc1_body(x_ref, w_ref, dg_ref, g_ref, dinv_ref):
    dinv = _dinv_from(dg_ref)
    h = jnp.dot(x_ref[...], w_ref[...], preferred_element_type=F32)
    g_ref[...] = h * dinv
    dinv_ref[...] = dinv


def _tc2_body(g_in_ref, p_ref, dg_ref, b_ref, w_ref, out_ref):
    dinv = _dinv_from(dg_ref)
    su = p_ref[0] + p_ref[1] + g_in_ref[...]
    h = jnp.maximum(dinv * su + b_ref[...], 0.0)
    out_ref[...] = dinv * jnp.dot(h, w_ref[...], preferred_element_type=F32)


def _tc3_body(g_in_ref, p_ref, dg_ref, wp_ref, b2_ref, w3_ref, b3_ref,
              out_ref, acc):
    i = pl.program_id(0)

    @pl.when(i == 0)
    def _():
        acc[...] = jnp.zeros((8, D), F32)

    dinv = _dinv_from(dg_ref)
    su = p_ref[0] + p_ref[1] + g_in_ref[...]
    h3 = jnp.maximum(dinv * su + b2_ref[...], 0.0)
    v = dinv * (wp_ref[0] + wp_ref[1] + dinv)
    acc[0:1, :] = acc[0:1, :] + jnp.sum(v * h3, axis=0, keepdims=True)

    @pl.when(i == pl.num_programs(0) - 1)
    def _():
        out_ref[...] = (
            jnp.dot(acc[0:1, :] / float(N), w3_ref[...],
                    preferred_element_type=F32) + b3_ref[...]
        )


def _row_spec(block):
    return pl.BlockSpec(block, lambda i: (i, 0))


def _const_spec(block):
    return pl.BlockSpec(block, lambda i: (0, 0))


_p_spec = pl.BlockSpec((2, TB, D), lambda i: (0, i, 0))
_v_spec = pl.BlockSpec((2, TB, 1), lambda i: (0, i, 0))

_tc1 = pl.pallas_call(
    _tc1_body,
    grid=(N // TB,),
    in_specs=[_row_spec((TB, D)), _const_spec((D, D)), _v_spec],
    out_specs=[_row_spec((TB, D)), _row_spec((TB, 1))],
    out_shape=[jax.ShapeDtypeStruct((N, D), F32),
               jax.ShapeDtypeStruct((N, 1), F32)],
)

_tc2 = pl.pallas_call(
    _tc2_body,
    grid=(N // TB,),
    in_specs=[_row_spec((TB, D)), _p_spec, _v_spec,
              _const_spec((1, D)), _const_spec((D, D))],
    out_specs=_row_spec((TB, D)),
    out_shape=jax.ShapeDtypeStruct((N, D), F32),
)

_tc3 = pl.pallas_call(
    _tc3_body,
    grid=(N // TB,),
    in_specs=[_row_spec((TB, D)), _p_spec, _v_spec, _v_spec,
              _const_spec((1, D)), _const_spec((D, D)), _const_spec((1, D))],
    out_specs=_const_spec((1, D)),
    out_shape=jax.ShapeDtypeStruct((1, D), F32),
    scratch_shapes=[pltpu.VMEM((8, D), F32)],
)


def kernel(x, edge_index, W1, b1, W2, b2, W3, b3):
    row = edge_index[0]
    col = edge_index[1]

    pad = NW * EPW - E
    pr = jnp.arange(pad, dtype=jnp.int32)
    roww = jnp.concatenate([row, pr % N]).reshape(NW, CW, K)
    colw = jnp.concatenate([col, N + pr % (NPAD2 - N)]).reshape(NW, CW, K)
    idxp = jnp.stack([roww, colw], axis=2)

    degp = _deg(colw).reshape(NC, NPAD, 1)

    g1, dinv = _tc1(x, W1, degp)
    dinv_pad = jnp.concatenate([dinv[:, 0], jnp.zeros((NPAD - N,), F32)])

    p1 = _agg(g1, idxp)
    g2 = _tc2(g1, p1, degp, b1.reshape(1, D), W2)
    p2, wp = _aggw(g2, idxp, dinv_pad)

    return _tc3(g2, p2, degp, wp.reshape(NC, NPAD, 1),
                b2.reshape(1, D), W3, b3.reshape(1, D))
